# baseline (speedup 1.0000x reference)
import functools

import jax
import jax.numpy as jnp
from jax import lax
from jax.experimental import pallas as pl
from jax.experimental.pallas import tpu as pltpu
from jax.experimental.pallas import tpu_sc as plsc

SEQ = 200
DIM = 128
BATCH = 1024
NC = 2
NS = 16
NW = NC * NS
B_TOTAL = BATCH * SEQ
B_PER_W = B_TOTAL // NW
CHUNK = SEQ
N_CHUNKS = B_PER_W // CHUNK
LANES = 16
VECS_PER_ROW = DIM // LANES


def _sc_body(idx_hbm, table_hbm, pos_hbm, out_hbm,
             idx_v, pos_v, buf0, buf1, buf2,
             gsem0, gsem1, gsem2, ssem0, ssem1, ssem2, psem):
  wid = lax.axis_index("s") * NC + lax.axis_index("c")
  base = wid * B_PER_W

  pltpu.sync_copy(idx_hbm.at[pl.ds(base, B_PER_W)], idx_v)
  pos_copy = pltpu.async_copy(pos_hbm, pos_v, psem)

  bufs = (buf0, buf1, buf2)
  gsems = (gsem0, gsem1, gsem2)
  ssems = (ssem0, ssem1, ssem2)

  def start_gather(g, b):
    return pltpu.async_copy(
        table_hbm.at[idx_v.at[pl.ds(g * CHUNK, CHUNK)]], bufs[b], gsems[b])

  PARTS = 5
  PART_ROWS = CHUNK // PARTS

  def add_and_scatter(g, b):
    handles = []
    for part in range(PARTS):
      r0 = part * PART_ROWS

      def _row(r, carry):
        for k in range(VECS_PER_ROW):
          sl = pl.ds(k * LANES, LANES)
          plsc.addupdate(bufs[b].at[r, sl], pos_v[r, sl])
        return carry
      lax.fori_loop(r0, r0 + PART_ROWS, _row, 0)

      handles.append(pltpu.async_copy(
          bufs[b].at[pl.ds(r0, PART_ROWS)],
          out_hbm.at[pl.ds(base + g * CHUNK + r0, PART_ROWS)],
          ssems[b]))
    return handles

  gh = [None] * N_CHUNKS
  sh = [None] * N_CHUNKS
  gh[0] = start_gather(0, 0)
  gh[1] = start_gather(1, 1)
  pos_copy.wait()
  for g in range(N_CHUNKS):
    b = g % 3
    gh[g].wait()
    if g + 2 < N_CHUNKS:
      if g >= 1:
        for h in sh[g - 1]:
          h.wait()
      gh[g + 2] = start_gather(g + 2, (g + 2) % 3)
    sh[g] = add_and_scatter(g, b)
  for g in (N_CHUNKS - 3, N_CHUNKS - 2, N_CHUNKS - 1):
    for h in sh[g]:
      h.wait()


@jax.jit
def _run(idx_flat, table, pos):
  kern = pl.kernel(
      _sc_body,
      out_type=jax.ShapeDtypeStruct((B_TOTAL, DIM), jnp.float32),
      mesh=plsc.VectorSubcoreMesh(
          core_axis_name="c", subcore_axis_name="s",
          num_cores=NC, num_subcores=NS),
      scratch_types=[
          pltpu.VMEM((B_PER_W,), jnp.int32),
          pltpu.VMEM((SEQ, DIM), jnp.float32),
          pltpu.VMEM((CHUNK, DIM), jnp.float32),
          pltpu.VMEM((CHUNK, DIM), jnp.float32),
          pltpu.VMEM((CHUNK, DIM), jnp.float32),
          pltpu.SemaphoreType.DMA,
          pltpu.SemaphoreType.DMA,
          pltpu.SemaphoreType.DMA,
          pltpu.SemaphoreType.DMA,
          pltpu.SemaphoreType.DMA,
          pltpu.SemaphoreType.DMA,
          pltpu.SemaphoreType.DMA,
      ],
  )
  return kern(idx_flat, table, pos)


def kernel(inputs, input_embedding_matrix, position_embedding_matrix):
  idx_flat = inputs.reshape(B_TOTAL)
  out = _run(idx_flat, input_embedding_matrix, position_embedding_matrix)
  return out.reshape(BATCH, SEQ, DIM)
